# baseline (speedup 1.0000x reference)
import functools

import jax
import jax.numpy as jnp
from jax import lax
from jax.experimental import pallas as pl
from jax.experimental.pallas import tpu as pltpu
from jax.experimental.pallas import tpu_sc as plsc

N = 10000
E = 640000
D_IN = 128
HID = 16
D_OUT = 128

NW = 32
EB = 100
ROWS = E // EB
RPT = ROWS // NW
RCH = 10
NCH = RPT // RCH
EPT = RPT * EB
CL = 2000
CNC = EPT // CL


_MESH = plsc.VectorSubcoreMesh(core_axis_name="c", subcore_axis_name="s")
_SC_PARAMS = pltpu.CompilerParams(use_tc_tiling_on_sc=False)


@functools.partial(
    pl.kernel,
    mesh=_MESH,
    out_type=jax.ShapeDtypeStruct((2, N, HID), jnp.float32),
    scratch_types=[
        pltpu.VMEM((RPT, EB), jnp.int32),
        pltpu.VMEM((EB, HID), jnp.float32),
        pltpu.VMEM_SHARED((N, HID), jnp.float32),
        pltpu.SemaphoreType.DMA,
        pltpu.SemaphoreType.DMA,
        pltpu.SemaphoreType.DMA,
    ],
    compiler_params=_SC_PARAMS,
)
def _deg_kernel(dst_hbm, zeros_hbm, ones_hbm, out_hbm, idxd, onesv, accd,
                isem, ssem0, ssem1):
    c = lax.axis_index("c")
    s = lax.axis_index("s")
    wid = s * 2 + c
    icp = pltpu.async_copy(dst_hbm.at[pl.ds(wid * RPT, RPT)], idxd, isem)
    pltpu.sync_copy(ones_hbm, onesv)

    nsl = N // 16
    sl = pl.ds(s * nsl, nsl)
    pltpu.sync_copy(zeros_hbm.at[sl], accd.at[sl])
    plsc.subcore_barrier()
    icp.wait()
    ssems = (ssem0, ssem1)

    def body(g2, carry):
        for b in range(2):
            ch = g2 * 2 + b

            @pl.when(ch >= 2)
            def _():
                for j in range(RCH):
                    pltpu.make_async_copy(
                        onesv, accd.at[idxd.at[0]], ssems[b]).wait()

            for j in range(RCH):
                pltpu.async_copy(
                    onesv, accd.at[idxd.at[ch * RCH + j]], ssems[b],
                    add=True)
        return carry

    lax.fori_loop(0, NCH // 2, body, 0)
    for b in range(2):
        for j in range(RCH):
            pltpu.make_async_copy(onesv, accd.at[idxd.at[0]], ssems[b]).wait()
    plsc.subcore_barrier()
    pltpu.sync_copy(accd.at[sl], out_hbm.at[c, sl])


@functools.partial(
    pl.kernel,
    mesh=_MESH,
    out_type=jax.ShapeDtypeStruct((2, N, HID), jnp.float32),
    scratch_types=[
        pltpu.VMEM((EPT,), jnp.int32),
        pltpu.VMEM((EPT,), jnp.int32),
        pltpu.VMEM((2, CL, HID), jnp.float32),
        pltpu.VMEM_SHARED((N, HID), jnp.float32),
        pltpu.VMEM_SHARED((N, HID), jnp.float32),
        pltpu.SemaphoreType.DMA,
        pltpu.SemaphoreType.DMA,
        pltpu.SemaphoreType.DMA,
        pltpu.SemaphoreType.DMA,
        pltpu.SemaphoreType.DMA,
    ],
    compiler_params=_SC_PARAMS,
)
def _agg_kernel(hs_hbm, src_hbm, dst_hbm, zeros_hbm, out_hbm,
                idx_s, idx_d, rows, hs_s, acc,
                isem, gsem0, gsem1, ssem0, ssem1):
    c = lax.axis_index("c")
    s = lax.axis_index("s")
    wid = s * 2 + c
    ic0 = pltpu.async_copy(src_hbm.at[pl.ds(wid * EPT, EPT)], idx_s, isem)
    ic1 = pltpu.async_copy(dst_hbm.at[pl.ds(wid * EPT, EPT)], idx_d, isem)

    nsl = N // 16
    sl = pl.ds(s * nsl, nsl)
    pltpu.sync_copy(hs_hbm.at[sl], hs_s.at[sl])
    pltpu.sync_copy(zeros_hbm.at[sl], acc.at[sl])
    plsc.subcore_barrier()
    ic0.wait()
    ic1.wait()
    gsems = (gsem0, gsem1)
    ssems = (ssem0, ssem1)

    def body(g2, carry):
        for b in range(2):
            ch = g2 * 2 + b

            @pl.when(ch >= 2)
            def _():
                pltpu.make_async_copy(
                    rows.at[b], acc.at[idx_d.at[pl.ds(0, CL)]],
                    ssems[b]).wait()

            pltpu.async_copy(
                hs_s.at[idx_s.at[pl.ds(ch * CL, CL)]], rows.at[b],
                gsems[b]).wait()
            pltpu.async_copy(
                rows.at[b], acc.at[idx_d.at[pl.ds(ch * CL, CL)]],
                ssems[b], add=True)
        return carry

    lax.fori_loop(0, CNC // 2, body, 0)
    for b in range(2):
        pltpu.make_async_copy(
            rows.at[b], acc.at[idx_d.at[pl.ds(0, CL)]], ssems[b]).wait()
    plsc.subcore_barrier()
    pltpu.sync_copy(acc.at[sl], out_hbm.at[c, sl])


BN = 1000
GRID = N // BN


def _prep_body(x_ref, w1_ref, d0_ref, d1_ref, hs1_ref, dis_ref):
    deg = d0_ref[:, :1] + d1_ref[:, :1] + 1.0
    dis = lax.rsqrt(deg)
    h1 = jnp.dot(x_ref[...], w1_ref[...], preferred_element_type=jnp.float32)
    hs1_ref[...] = h1 * dis
    dis_ref[...] = dis


_prep_call = pl.pallas_call(
    _prep_body,
    grid=(GRID,),
    in_specs=[
        pl.BlockSpec((BN, D_IN), lambda i: (i, 0)),
        pl.BlockSpec((D_IN, HID), lambda i: (0, 0)),
        pl.BlockSpec((BN, HID), lambda i: (i, 0)),
        pl.BlockSpec((BN, HID), lambda i: (i, 0)),
    ],
    out_specs=[
        pl.BlockSpec((BN, HID), lambda i: (i, 0)),
        pl.BlockSpec((BN, 1), lambda i: (i, 0)),
    ],
    out_shape=[
        jax.ShapeDtypeStruct((N, HID), jnp.float32),
        jax.ShapeDtypeStruct((N, 1), jnp.float32),
    ],
)


def _mid_body(p0_ref, p1_ref, hs1_ref, dis_ref, b1_ref, hs2_ref):
    dis = dis_ref[...]
    t = (p0_ref[...] + p1_ref[...] + hs1_ref[...]) * dis + b1_ref[...]
    hs2_ref[...] = jnp.maximum(t, 0.0) * dis


_mid_call = pl.pallas_call(
    _mid_body,
    grid=(GRID,),
    in_specs=[
        pl.BlockSpec((BN, HID), lambda i: (i, 0)),
        pl.BlockSpec((BN, HID), lambda i: (i, 0)),
        pl.BlockSpec((BN, HID), lambda i: (i, 0)),
        pl.BlockSpec((BN, 1), lambda i: (i, 0)),
        pl.BlockSpec((1, HID), lambda i: (0, 0)),
    ],
    out_specs=pl.BlockSpec((BN, HID), lambda i: (i, 0)),
    out_shape=jax.ShapeDtypeStruct((N, HID), jnp.float32),
)


def _out_body(q0_ref, q1_ref, hs2_ref, dis_ref, w2_ref, b2_ref, o_ref):
    t = (q0_ref[...] + q1_ref[...] + hs2_ref[...]) * dis_ref[...]
    h = jnp.dot(t, w2_ref[...], preferred_element_type=jnp.float32) + b2_ref[...]
    m = jnp.max(h, axis=1, keepdims=True)
    ex = jnp.exp(h - m)
    ssum = jnp.sum(ex, axis=1, keepdims=True)
    o_ref[...] = (h - m) - jnp.log(ssum)


_out_call = pl.pallas_call(
    _out_body,
    grid=(GRID,),
    in_specs=[
        pl.BlockSpec((BN, HID), lambda i: (i, 0)),
        pl.BlockSpec((BN, HID), lambda i: (i, 0)),
        pl.BlockSpec((BN, HID), lambda i: (i, 0)),
        pl.BlockSpec((BN, 1), lambda i: (i, 0)),
        pl.BlockSpec((HID, D_OUT), lambda i: (0, 0)),
        pl.BlockSpec((1, D_OUT), lambda i: (0, 0)),
    ],
    out_specs=pl.BlockSpec((BN, D_OUT), lambda i: (i, 0)),
    out_shape=jax.ShapeDtypeStruct((N, D_OUT), jnp.float32),
)


def kernel(traffic, path_to_queue, W1, b1, W2, b2):
    src2d = path_to_queue[0].reshape(ROWS, EB)
    dst2d = path_to_queue[1].reshape(ROWS, EB)
    src1d = path_to_queue[0]
    dst1d = path_to_queue[1]
    zeros2d = jnp.zeros((N, HID), jnp.float32)
    onese = jnp.ones((EB, HID), jnp.float32)

    d = _deg_kernel(dst2d, zeros2d, onese)
    hs1, dis = _prep_call(traffic, W1, d[0], d[1])

    p = _agg_kernel(hs1, src1d, dst1d, zeros2d)
    hs2 = _mid_call(p[0], p[1], hs1, dis, b1.reshape(1, HID))

    q = _agg_kernel(hs2, src1d, dst1d, zeros2d)
    return _out_call(q[0], q[1], hs2, dis, W2, b2.reshape(1, D_OUT))


# speedup vs baseline: 1.0192x; 1.0192x over previous
import functools

import jax
import jax.numpy as jnp
from jax import lax
from jax.experimental import pallas as pl
from jax.experimental.pallas import tpu as pltpu
from jax.experimental.pallas import tpu_sc as plsc

N = 10000
E = 640000
D_IN = 128
HID = 16
D_OUT = 128

NW = 32
EB = 100
ROWS = E // EB
RPT = ROWS // NW
RCH = 10
NCH = RPT // RCH
EPT = RPT * EB
CL = 2000
CNC = EPT // CL


_MESH = plsc.VectorSubcoreMesh(core_axis_name="c", subcore_axis_name="s")
_SC_PARAMS = pltpu.CompilerParams(use_tc_tiling_on_sc=False)


@functools.partial(
    pl.kernel,
    mesh=_MESH,
    out_type=jax.ShapeDtypeStruct((2, N, HID), jnp.float32),
    scratch_types=[
        pltpu.VMEM((RPT, EB), jnp.int32),
        pltpu.VMEM((EB, HID), jnp.float32),
        pltpu.VMEM_SHARED((N, HID), jnp.float32),
        pltpu.SemaphoreType.DMA,
        pltpu.SemaphoreType.DMA,
        pltpu.SemaphoreType.DMA,
    ],
    compiler_params=_SC_PARAMS,
)
def _deg_kernel(dst_hbm, zeros_hbm, ones_hbm, out_hbm, idxd, onesv, accd,
                isem, ssem0, ssem1):
    c = lax.axis_index("c")
    s = lax.axis_index("s")
    wid = s * 2 + c
    icp = pltpu.async_copy(dst_hbm.at[pl.ds(wid * RPT, RPT)], idxd, isem)
    pltpu.sync_copy(ones_hbm, onesv)

    nsl = N // 16
    sl = pl.ds(s * nsl, nsl)
    pltpu.sync_copy(zeros_hbm.at[sl], accd.at[sl])
    plsc.subcore_barrier()
    icp.wait()
    ssems = (ssem0, ssem1)

    def body(g2, carry):
        for b in range(2):
            ch = g2 * 2 + b

            @pl.when(ch >= 2)
            def _():
                for j in range(RCH):
                    pltpu.make_async_copy(
                        onesv, accd.at[idxd.at[0]], ssems[b]).wait()

            for j in range(RCH):
                pltpu.async_copy(
                    onesv, accd.at[idxd.at[ch * RCH + j]], ssems[b],
                    add=True)
        return carry

    lax.fori_loop(0, NCH // 2, body, 0)
    for b in range(2):
        for j in range(RCH):
            pltpu.make_async_copy(onesv, accd.at[idxd.at[0]], ssems[b]).wait()
    plsc.subcore_barrier()
    pltpu.sync_copy(accd.at[sl], out_hbm.at[c, sl])


@functools.partial(
    pl.kernel,
    mesh=_MESH,
    out_type=jax.ShapeDtypeStruct((2, N, HID), jnp.float32),
    scratch_types=[
        pltpu.VMEM((EPT,), jnp.int32),
        pltpu.VMEM((EPT,), jnp.int32),
        pltpu.VMEM((2, CL, HID), jnp.float32),
        pltpu.VMEM_SHARED((N, HID), jnp.float32),
        pltpu.VMEM_SHARED((N, HID), jnp.float32),
        pltpu.SemaphoreType.DMA,
        pltpu.SemaphoreType.DMA,
        pltpu.SemaphoreType.DMA,
        pltpu.SemaphoreType.DMA,
        pltpu.SemaphoreType.DMA,
    ],
    compiler_params=_SC_PARAMS,
)
def _agg_kernel(hs_hbm, src_hbm, dst_hbm, zeros_hbm, out_hbm,
                idx_s, idx_d, rows, hs_s, acc,
                isem, gsem0, gsem1, ssem0, ssem1):
    c = lax.axis_index("c")
    s = lax.axis_index("s")
    wid = s * 2 + c
    ic0 = pltpu.async_copy(src_hbm.at[pl.ds(wid * EPT, EPT)], idx_s, isem)
    ic1 = pltpu.async_copy(dst_hbm.at[pl.ds(wid * EPT, EPT)], idx_d, isem)

    nsl = N // 16
    sl = pl.ds(s * nsl, nsl)
    pltpu.sync_copy(hs_hbm.at[sl], hs_s.at[sl])
    pltpu.sync_copy(zeros_hbm.at[sl], acc.at[sl])
    plsc.subcore_barrier()
    ic0.wait()
    ic1.wait()
    gsems = (gsem0, gsem1)
    ssems = (ssem0, ssem1)

    def body(g2, carry):
        for b in range(2):
            ch = g2 * 2 + b

            @pl.when(ch >= 2)
            def _():
                pltpu.make_async_copy(
                    rows.at[b], acc.at[idx_d.at[pl.ds(0, CL)]],
                    ssems[b]).wait()

            pltpu.async_copy(
                hs_s.at[idx_s.at[pl.ds(ch * CL, CL)]], rows.at[b],
                gsems[b]).wait()
            pltpu.async_copy(
                rows.at[b], acc.at[idx_d.at[pl.ds(ch * CL, CL)]],
                ssems[b], add=True)
        return carry

    lax.fori_loop(0, CNC // 2, body, 0)
    for b in range(2):
        pltpu.make_async_copy(
            rows.at[b], acc.at[idx_d.at[pl.ds(0, CL)]], ssems[b]).wait()
    plsc.subcore_barrier()
    pltpu.sync_copy(acc.at[sl], out_hbm.at[c, sl])


BN = 1000
GRID = N // BN


def _prep_body(x_ref, w1_ref, d0_ref, d1_ref, hs1_ref, dis_ref):
    deg = d0_ref[:, :1] + d1_ref[:, :1] + 1.0
    dis = lax.rsqrt(deg)
    h1 = jnp.dot(x_ref[...], w1_ref[...], preferred_element_type=jnp.float32)
    hs1_ref[...] = h1 * dis
    dis_ref[...] = dis


_prep_call = pl.pallas_call(
    _prep_body,
    out_shape=[
        jax.ShapeDtypeStruct((N, HID), jnp.float32),
        jax.ShapeDtypeStruct((N, 1), jnp.float32),
    ],
)


def _mid_body(p0_ref, p1_ref, hs1_ref, dis_ref, b1_ref, hs2_ref):
    dis = dis_ref[...]
    t = (p0_ref[...] + p1_ref[...] + hs1_ref[...]) * dis + b1_ref[...]
    hs2_ref[...] = jnp.maximum(t, 0.0) * dis


_mid_call = pl.pallas_call(
    _mid_body,
    out_shape=jax.ShapeDtypeStruct((N, HID), jnp.float32),
)


def _out_body(q0_ref, q1_ref, hs2_ref, dis_ref, w2_ref, b2_ref, o_ref):
    t = (q0_ref[...] + q1_ref[...] + hs2_ref[...]) * dis_ref[...]
    h = jnp.dot(t, w2_ref[...], preferred_element_type=jnp.float32) + b2_ref[...]
    m = jnp.max(h, axis=1, keepdims=True)
    ex = jnp.exp(h - m)
    ssum = jnp.sum(ex, axis=1, keepdims=True)
    o_ref[...] = (h - m) - jnp.log(ssum)


_out_call = pl.pallas_call(
    _out_body,
    out_shape=jax.ShapeDtypeStruct((N, D_OUT), jnp.float32),
)


def kernel(traffic, path_to_queue, W1, b1, W2, b2):
    src2d = path_to_queue[0].reshape(ROWS, EB)
    dst2d = path_to_queue[1].reshape(ROWS, EB)
    src1d = path_to_queue[0]
    dst1d = path_to_queue[1]
    zeros2d = jnp.zeros((N, HID), jnp.float32)
    onese = jnp.ones((EB, HID), jnp.float32)

    d = _deg_kernel(dst2d, zeros2d, onese)
    hs1, dis = _prep_call(traffic, W1, d[0], d[1])

    p = _agg_kernel(hs1, src1d, dst1d, zeros2d)
    hs2 = _mid_call(p[0], p[1], hs1, dis, b1.reshape(1, HID))

    q = _agg_kernel(hs2, src1d, dst1d, zeros2d)
    return _out_call(q[0], q[1], hs2, dis, W2, b2.reshape(1, D_OUT))


# speedup vs baseline: 1.0413x; 1.0216x over previous
import functools

import jax
import jax.numpy as jnp
from jax import lax
from jax.experimental import pallas as pl
from jax.experimental.pallas import tpu as pltpu
from jax.experimental.pallas import tpu_sc as plsc

N = 10000
E = 640000
D_IN = 128
HID = 16
D_OUT = 128

NW = 32
EB = 100
ROWS = E // EB
RPT = ROWS // NW
RCH = 10
NCH = RPT // RCH
EPT = RPT * EB
CL = 1000
CNC = EPT // CL


_MESH = plsc.VectorSubcoreMesh(core_axis_name="c", subcore_axis_name="s")
_SC_PARAMS = pltpu.CompilerParams(use_tc_tiling_on_sc=False)


@functools.partial(
    pl.kernel,
    mesh=_MESH,
    out_type=jax.ShapeDtypeStruct((2, N, HID), jnp.float32),
    scratch_types=[
        pltpu.VMEM((RPT, EB), jnp.int32),
        pltpu.VMEM((EB, HID), jnp.float32),
        pltpu.VMEM_SHARED((N, HID), jnp.float32),
        pltpu.SemaphoreType.DMA,
        pltpu.SemaphoreType.DMA,
        pltpu.SemaphoreType.DMA,
    ],
    compiler_params=_SC_PARAMS,
)
def _deg_kernel(dst_hbm, zeros_hbm, ones_hbm, out_hbm, idxd, onesv, accd,
                isem, ssem0, ssem1):
    c = lax.axis_index("c")
    s = lax.axis_index("s")
    wid = s * 2 + c
    icp = pltpu.async_copy(dst_hbm.at[pl.ds(wid * RPT, RPT)], idxd, isem)
    pltpu.sync_copy(ones_hbm, onesv)

    nsl = N // 16
    sl = pl.ds(s * nsl, nsl)
    pltpu.sync_copy(zeros_hbm.at[sl], accd.at[sl])
    plsc.subcore_barrier()
    icp.wait()
    ssems = (ssem0, ssem1)

    def body(g2, carry):
        for b in range(2):
            ch = g2 * 2 + b

            @pl.when(ch >= 2)
            def _():
                for j in range(RCH):
                    pltpu.make_async_copy(
                        onesv, accd.at[idxd.at[0]], ssems[b]).wait()

            for j in range(RCH):
                pltpu.async_copy(
                    onesv, accd.at[idxd.at[ch * RCH + j]], ssems[b],
                    add=True)
        return carry

    lax.fori_loop(0, NCH // 2, body, 0)
    for b in range(2):
        for j in range(RCH):
            pltpu.make_async_copy(onesv, accd.at[idxd.at[0]], ssems[b]).wait()
    plsc.subcore_barrier()
    pltpu.sync_copy(accd.at[sl], out_hbm.at[c, sl])


@functools.partial(
    pl.kernel,
    mesh=_MESH,
    out_type=jax.ShapeDtypeStruct((2, N, HID), jnp.float32),
    scratch_types=[
        pltpu.VMEM((EPT,), jnp.int32),
        pltpu.VMEM((EPT,), jnp.int32),
        pltpu.VMEM((2, CL, HID), jnp.float32),
        pltpu.VMEM_SHARED((N, HID), jnp.float32),
        pltpu.VMEM_SHARED((N, HID), jnp.float32),
        pltpu.SemaphoreType.DMA,
        pltpu.SemaphoreType.DMA,
        pltpu.SemaphoreType.DMA,
        pltpu.SemaphoreType.DMA,
        pltpu.SemaphoreType.DMA,
    ],
    compiler_params=_SC_PARAMS,
)
def _agg_kernel(hs_hbm, src_hbm, dst_hbm, zeros_hbm, out_hbm,
                idx_s, idx_d, rows, hs_s, acc,
                isem, gsem0, gsem1, ssem0, ssem1):
    c = lax.axis_index("c")
    s = lax.axis_index("s")
    wid = s * 2 + c
    ic0 = pltpu.async_copy(src_hbm.at[pl.ds(wid * EPT, EPT)], idx_s, isem)
    ic1 = pltpu.async_copy(dst_hbm.at[pl.ds(wid * EPT, EPT)], idx_d, isem)

    nsl = N // 16
    sl = pl.ds(s * nsl, nsl)
    pltpu.sync_copy(hs_hbm.at[sl], hs_s.at[sl])
    pltpu.sync_copy(zeros_hbm.at[sl], acc.at[sl])
    plsc.subcore_barrier()
    ic0.wait()
    ic1.wait()
    gsems = (gsem0, gsem1)
    ssems = (ssem0, ssem1)

    def body(g2, carry):
        for b in range(2):
            ch = g2 * 2 + b

            @pl.when(ch >= 2)
            def _():
                pltpu.make_async_copy(
                    rows.at[b], acc.at[idx_d.at[pl.ds(0, CL)]],
                    ssems[b]).wait()

            pltpu.async_copy(
                hs_s.at[idx_s.at[pl.ds(ch * CL, CL)]], rows.at[b],
                gsems[b]).wait()
            pltpu.async_copy(
                rows.at[b], acc.at[idx_d.at[pl.ds(ch * CL, CL)]],
                ssems[b], add=True)
        return carry

    lax.fori_loop(0, CNC // 2, body, 0)
    for b in range(2):
        pltpu.make_async_copy(
            rows.at[b], acc.at[idx_d.at[pl.ds(0, CL)]], ssems[b]).wait()
    plsc.subcore_barrier()
    pltpu.sync_copy(acc.at[sl], out_hbm.at[c, sl])


NSL = N // 16
MCH = 125
MNC = NSL // MCH


@functools.partial(
    pl.kernel,
    mesh=_MESH,
    out_type=[
        jax.ShapeDtypeStruct((2, N, HID), jnp.float32),
        jax.ShapeDtypeStruct((N, HID), jnp.float32),
    ],
    scratch_types=[
        pltpu.VMEM((EPT,), jnp.int32),
        pltpu.VMEM((EPT,), jnp.int32),
        pltpu.VMEM((2, CL, HID), jnp.float32),
        pltpu.VMEM((MCH, HID), jnp.float32),
        pltpu.VMEM((MCH, HID), jnp.float32),
        pltpu.VMEM((MCH, HID), jnp.float32),
        pltpu.VMEM((MCH, HID), jnp.float32),
        pltpu.VMEM((MCH, HID), jnp.float32),
        pltpu.VMEM((HID,), jnp.float32),
        pltpu.VMEM_SHARED((N, HID), jnp.float32),
        pltpu.VMEM_SHARED((N, HID), jnp.float32),
        pltpu.SemaphoreType.DMA,
        pltpu.SemaphoreType.DMA,
        pltpu.SemaphoreType.DMA,
        pltpu.SemaphoreType.DMA,
        pltpu.SemaphoreType.DMA,
    ],
    compiler_params=_SC_PARAMS,
)
def _agg2_kernel(p_hbm, hs1_hbm, dis_hbm, b1_hbm, src_hbm, dst_hbm, zeros_hbm,
                 out_hbm, hs2_hbm,
                 idx_s, idx_d, rows, p0v, p1v, h1v, dsv, h2v, b1v,
                 hs_s, acc, isem, gsem0, gsem1, ssem0, ssem1):
    c = lax.axis_index("c")
    s = lax.axis_index("s")
    wid = s * 2 + c
    ic0 = pltpu.async_copy(src_hbm.at[pl.ds(wid * EPT, EPT)], idx_s, isem)
    ic1 = pltpu.async_copy(dst_hbm.at[pl.ds(wid * EPT, EPT)], idx_d, isem)
    pltpu.sync_copy(b1_hbm, b1v)
    nsl = N // 16
    sl = pl.ds(s * nsl, nsl)
    pltpu.sync_copy(zeros_hbm.at[sl], acc.at[sl])

    def mid_chunk(m, carry):
        base = s * nsl + m * MCH
        msl = pl.ds(base, MCH)
        pltpu.sync_copy(p_hbm.at[0, msl], p0v)
        pltpu.sync_copy(p_hbm.at[1, msl], p1v)
        pltpu.sync_copy(hs1_hbm.at[msl], h1v)
        pltpu.sync_copy(dis_hbm.at[msl], dsv)
        bias = b1v[...]

        def row(r, carry2):
            dis = dsv[r]
            t = (p0v[r] + p1v[r] + h1v[r]) * dis + bias
            h2v[r] = jnp.maximum(t, 0.0) * dis
            return carry2

        lax.fori_loop(0, MCH, row, 0)
        pltpu.sync_copy(h2v, hs_s.at[msl])
        pltpu.sync_copy(h2v, hs2_hbm.at[msl])
        return carry

    lax.fori_loop(0, MNC, mid_chunk, 0)
    plsc.subcore_barrier()
    ic0.wait()
    ic1.wait()
    gsems = (gsem0, gsem1)
    ssems = (ssem0, ssem1)

    def body(g2, carry):
        for b in range(2):
            ch = g2 * 2 + b

            @pl.when(ch >= 2)
            def _():
                pltpu.make_async_copy(
                    rows.at[b], acc.at[idx_d.at[pl.ds(0, CL)]],
                    ssems[b]).wait()

            pltpu.async_copy(
                hs_s.at[idx_s.at[pl.ds(ch * CL, CL)]], rows.at[b],
                gsems[b]).wait()
            pltpu.async_copy(
                rows.at[b], acc.at[idx_d.at[pl.ds(ch * CL, CL)]],
                ssems[b], add=True)
        return carry

    lax.fori_loop(0, CNC // 2, body, 0)
    for b in range(2):
        pltpu.make_async_copy(
            rows.at[b], acc.at[idx_d.at[pl.ds(0, CL)]], ssems[b]).wait()
    plsc.subcore_barrier()
    pltpu.sync_copy(acc.at[sl], out_hbm.at[c, sl])


BN = 1000
GRID = N // BN


def _prep_body(x_ref, w1_ref, d0_ref, d1_ref, hs1_ref, dis_ref):
    deg = d0_ref[:, :1] + d1_ref[:, :1] + 1.0
    dis = lax.rsqrt(deg)
    h1 = jnp.dot(x_ref[...], w1_ref[...], preferred_element_type=jnp.float32)
    hs1_ref[...] = h1 * dis
    dis_ref[...] = jnp.broadcast_to(dis, (N, HID))


_prep_call = pl.pallas_call(
    _prep_body,
    out_shape=[
        jax.ShapeDtypeStruct((N, HID), jnp.float32),
        jax.ShapeDtypeStruct((N, HID), jnp.float32),
    ],
)


def _out_body(q0_ref, q1_ref, hs2_ref, dis_ref, w2_ref, b2_ref, o_ref):
    t = (q0_ref[...] + q1_ref[...] + hs2_ref[...]) * dis_ref[...]
    h = jnp.dot(t, w2_ref[...], preferred_element_type=jnp.float32) + b2_ref[...]
    m = jnp.max(h, axis=1, keepdims=True)
    ex = jnp.exp(h - m)
    ssum = jnp.sum(ex, axis=1, keepdims=True)
    o_ref[...] = (h - m) - jnp.log(ssum)


_out_call = pl.pallas_call(
    _out_body,
    out_shape=jax.ShapeDtypeStruct((N, D_OUT), jnp.float32),
)


def kernel(traffic, path_to_queue, W1, b1, W2, b2):
    src2d = path_to_queue[0].reshape(ROWS, EB)
    dst2d = path_to_queue[1].reshape(ROWS, EB)
    src1d = path_to_queue[0]
    dst1d = path_to_queue[1]
    zeros2d = jnp.zeros((N, HID), jnp.float32)
    onese = jnp.ones((EB, HID), jnp.float32)

    d = _deg_kernel(dst2d, zeros2d, onese)
    hs1, dis = _prep_call(traffic, W1, d[0], d[1])

    p = _agg_kernel(hs1, src1d, dst1d, zeros2d)
    q, hs2 = _agg2_kernel(p, hs1, dis, b1, src1d, dst1d, zeros2d)
    return _out_call(q[0], q[1], hs2, dis, W2, b2.reshape(1, D_OUT))
